# baseline (speedup 1.0000x reference)
import functools

import jax
import jax.numpy as jnp
from jax import lax
from jax.experimental import pallas as pl
from jax.experimental.pallas import tpu as pltpu
from jax.experimental.pallas import tpu_sc as plsc

N = 8192
NBLK = 8
BLKN = N // NBLK
M = 262144
C2 = 25.0
W = 32
NC = 2
ROWS = N // W
WPB = BLKN // ROWS
CAP = 24576
OUT_PER_W = M // W
LANES = 16


def _two_sum(a, b):
    s = a + b
    bp = s - a
    e = (a - (s - bp)) + (b - bp)
    return s, e


def _splat(ref, i):
    return plsc.load_gather(ref, [jnp.broadcast_to(i, (LANES,))])


ROW_BOUNDS = (0, 138, 301, 513, 1024)


def _phase1_body(xs, ys, zs, sq, stage_pd, stage_d12, counts,
                 xv, yv, zv, sqv, bpd, bd12, cnt_v):
    cid = lax.axis_index("c")
    sid = lax.axis_index("s")
    w = sid * NC + cid
    blk = w // WPB
    k = w % WPB
    base_li = jnp.where(
        k == 0, ROW_BOUNDS[0],
        jnp.where(k == 1, ROW_BOUNDS[1],
                  jnp.where(k == 2, ROW_BOUNDS[2], ROW_BOUNDS[3])))
    end_li = jnp.where(
        k == 0, ROW_BOUNDS[1],
        jnp.where(k == 1, ROW_BOUNDS[2],
                  jnp.where(k == 2, ROW_BOUNDS[3], ROW_BOUNDS[4])))
    gbase = blk * BLKN
    pltpu.sync_copy(xs.at[pl.ds(gbase, BLKN)], xv)
    pltpu.sync_copy(ys.at[pl.ds(gbase, BLKN)], yv)
    pltpu.sync_copy(zs.at[pl.ds(gbase, BLKN)], zv)
    pltpu.sync_copy(sq.at[pl.ds(gbase, BLKN)], sqv)
    lanes = lax.iota(jnp.int32, LANES)

    def row_body(li, cursor):
        xi = _splat(xv, li)
        yi = _splat(yv, li)
        zi = _splat(zv, li)
        sqi = _splat(sqv, li)
        gi = gbase + li
        gi_hi = jnp.broadcast_to(gi * 8192, (LANES,))
        t0 = (li // LANES) & ~3

        def t_batch(tb, cur):
            base = jnp.minimum(cur, CAP - 4 * LANES)
            ovec = jnp.broadcast_to(base, (LANES,))
            offs_k = base
            for k in range(4):
                t = t0 + tb * 4 + k
                jv = lanes + t * LANES
                xj = xv[pl.ds(t * LANES, LANES)]
                yj = yv[pl.ds(t * LANES, LANES)]
                zj = zv[pl.ds(t * LANES, LANES)]
                sqj = sqv[pl.ds(t * LANES, LANES)]
                xx = xi * xj
                yy = yi * yj
                zz = zi * zj
                s1, e1 = _two_sum(xx, yy)
                s2, e2 = _two_sum(s1, zz)
                mm = s2 + (e1 + e2)
                d = (sqi + sqj) - 2.0 * mm
                m = (d < C2) & (jv > li)
                plsc.store_compressed(bpd.at[pl.ds(offs_k, LANES)],
                                      gi_hi + (jv + gbase), mask=m)
                plsc.store_compressed(bd12.at[pl.ds(offs_k, LANES)],
                                      jnp.maximum(d, 0.0), mask=m)
                ovec = ovec + plsc.all_reduce_population_count(m)
                if k < 3:
                    offs_k = ovec[0]
            return ovec[0]

        return lax.fori_loop(0, (BLKN // LANES - t0) // 4, t_batch, cursor)

    cursor = lax.fori_loop(base_li, end_li, row_body, jnp.int32(0))
    cnt_v[...] = jnp.broadcast_to(cursor, (LANES,))
    pltpu.sync_copy(cnt_v, counts.at[w])
    pltpu.sync_copy(bpd, stage_pd.at[w])
    pltpu.sync_copy(bd12, stage_d12.at[w])


CH = 2048


def _phase2_body(spd, sd12, counts, esrc, edst, ed12,
                 cnt_vm, tpd, td12, gpd, gd12, osrc, odst, od12):
    cid = lax.axis_index("c")
    sid = lax.axis_index("s")
    w = sid * NC + cid
    pbase = w * OUT_PER_W
    pltpu.sync_copy(counts, cnt_vm)
    lanes = lax.iota(jnp.int32, LANES)
    zeros = jnp.zeros((LANES,), jnp.int32)

    total_v = zeros
    offs = []
    for s in range(W):
        cs = plsc.load_gather(cnt_vm, [jnp.full((LANES,), s, jnp.int32), zeros])
        offs.append(total_v[0])
        total_v = total_v + cs
    total = total_v[0]
    pe = jnp.minimum(total, pbase + OUT_PER_W)

    for s in range(W):
        a = jnp.maximum(offs[s], pbase)
        b = jnp.minimum(offs[s + 1] if s + 1 < W else total, pe)

        @pl.when(b > a)
        def _copy(s=s, a=a, b=b):
            srcoff = a - offs[s]
            sal = srcoff & ~7
            shift = srcoff - sal
            ln = (b - a) + shift
            base_flat = s * CAP + sal

            def chunk(cc, _):
                boff = pl.multiple_of(base_flat + cc * CH, 8)
                pltpu.sync_copy(spd.at[pl.ds(boff, CH)],
                                tpd.at[pl.ds(cc * CH, CH)])
                pltpu.sync_copy(sd12.at[pl.ds(boff, CH)],
                                td12.at[pl.ds(cc * CH, CH)])
                return 0

            lax.fori_loop(0, (ln + CH - 1) // CH, chunk, 0)
            dbase = a - pbase

            def shiftcp(v, _):
                gpd[pl.ds(dbase + v * LANES, LANES)] = (
                    tpd[pl.ds(shift + v * LANES, LANES)])
                gd12[pl.ds(dbase + v * LANES, LANES)] = (
                    td12[pl.ds(shift + v * LANES, LANES)])
                return 0

            lax.fori_loop(0, (b - a + LANES - 1) // LANES, shiftcp, 0)

    def post(t, _):
        sl = pl.ds(t * LANES, LANES)
        p = lanes + (pbase + t * LANES)
        v = p < total
        pd = gpd[sl]
        osrc[sl] = jnp.where(v, pd >> 13, N)
        odst[sl] = jnp.where(v, pd & 8191, N)
        od12[sl] = jnp.where(v, gd12[sl], C2)
        return 0

    lax.fori_loop(0, OUT_PER_W // LANES, post, 0)
    pltpu.sync_copy(osrc, esrc.at[pl.ds(pbase, OUT_PER_W)])
    pltpu.sync_copy(odst, esrc.at[pl.ds(M + pbase, OUT_PER_W)])
    pltpu.sync_copy(odst, edst.at[pl.ds(pbase, OUT_PER_W)])
    pltpu.sync_copy(osrc, edst.at[pl.ds(M + pbase, OUT_PER_W)])
    pltpu.sync_copy(od12, ed12.at[pl.ds(pbase, OUT_PER_W)])
    pltpu.sync_copy(od12, ed12.at[pl.ds(M + pbase, OUT_PER_W)])


def _bf16_round(x):
    y = lax.bitcast_convert_type(x, jnp.int32)
    r = (y + 0x7FFF + ((y >> 16) & 1)) & ~0xFFFF
    return lax.bitcast_convert_type(r, jnp.float32)


@functools.partial(jax.jit, static_argnames=())
def kernel(coordinates, batch_index, natoms):
    del batch_index, natoms
    c = coordinates
    sq = jnp.sum(c * c, axis=-1)
    ch = _bf16_round(c)
    xs = ch[:, 0]
    ys = ch[:, 1]
    zs = ch[:, 2]

    mesh = plsc.VectorSubcoreMesh(core_axis_name="c", subcore_axis_name="s")
    spd, sd12, counts = pl.kernel(
        _phase1_body,
        out_type=[
            jax.ShapeDtypeStruct((W + 1, CAP), jnp.int32),
            jax.ShapeDtypeStruct((W + 1, CAP), jnp.float32),
            jax.ShapeDtypeStruct((W, LANES), jnp.int32),
        ],
        mesh=mesh,
        compiler_params=pltpu.CompilerParams(needs_layout_passes=False),
        scratch_types=[
            pltpu.VMEM((BLKN,), jnp.float32),
            pltpu.VMEM((BLKN,), jnp.float32),
            pltpu.VMEM((BLKN,), jnp.float32),
            pltpu.VMEM((BLKN,), jnp.float32),
            pltpu.VMEM((CAP,), jnp.int32),
            pltpu.VMEM((CAP,), jnp.float32),
            pltpu.VMEM((LANES,), jnp.int32),
        ],
    )(xs, ys, zs, sq)

    esrc, edst, ed12 = pl.kernel(
        _phase2_body,
        out_type=[
            jax.ShapeDtypeStruct((2 * M,), jnp.int32),
            jax.ShapeDtypeStruct((2 * M,), jnp.int32),
            jax.ShapeDtypeStruct((2 * M,), jnp.float32),
        ],
        mesh=mesh,
        compiler_params=pltpu.CompilerParams(needs_layout_passes=False),
        scratch_types=[
            pltpu.VMEM((W, LANES), jnp.int32),
            pltpu.VMEM((5 * CH,), jnp.int32),
            pltpu.VMEM((5 * CH,), jnp.float32),
            pltpu.VMEM((OUT_PER_W + LANES,), jnp.int32),
            pltpu.VMEM((OUT_PER_W + LANES,), jnp.float32),
            pltpu.VMEM((OUT_PER_W,), jnp.int32),
            pltpu.VMEM((OUT_PER_W,), jnp.int32),
            pltpu.VMEM((OUT_PER_W,), jnp.float32),
        ],
    )(spd.reshape(-1), sd12.reshape(-1), counts)

    ie = jnp.arange(M, dtype=jnp.int32)
    isym = jnp.concatenate((ie + M, ie))
    return esrc, edst, ed12, isym


# speedup vs baseline: 1.0473x; 1.0473x over previous
import functools

import jax
import jax.numpy as jnp
from jax import lax
from jax.experimental import pallas as pl
from jax.experimental.pallas import tpu as pltpu
from jax.experimental.pallas import tpu_sc as plsc

N = 8192
NBLK = 8
BLKN = N // NBLK
M = 262144
C2 = 25.0
W = 32
NC = 2
ROWS = N // W
WPB = BLKN // ROWS
CAP = 24576
OUT_PER_W = M // W
LANES = 16


def _two_sum(a, b):
    s = a + b
    bp = s - a
    e = (a - (s - bp)) + (b - bp)
    return s, e


def _splat(ref, i):
    return plsc.load_gather(ref, [jnp.broadcast_to(i, (LANES,))])


ROW_BOUNDS = (0, 138, 301, 513, 1024)


def _phase1_body(xs, ys, zs, sq, stage_pd, stage_d12, counts,
                 xv, yv, zv, sqv, bpd, bd12, cnt_v):
    cid = lax.axis_index("c")
    sid = lax.axis_index("s")
    w = sid * NC + cid
    blk = w // WPB
    k = w % WPB
    base_li = jnp.where(
        k == 0, ROW_BOUNDS[0],
        jnp.where(k == 1, ROW_BOUNDS[1],
                  jnp.where(k == 2, ROW_BOUNDS[2], ROW_BOUNDS[3])))
    end_li = jnp.where(
        k == 0, ROW_BOUNDS[1],
        jnp.where(k == 1, ROW_BOUNDS[2],
                  jnp.where(k == 2, ROW_BOUNDS[3], ROW_BOUNDS[4])))
    gbase = blk * BLKN
    pltpu.sync_copy(xs.at[pl.ds(gbase, BLKN)], xv)
    pltpu.sync_copy(ys.at[pl.ds(gbase, BLKN)], yv)
    pltpu.sync_copy(zs.at[pl.ds(gbase, BLKN)], zv)
    pltpu.sync_copy(sq.at[pl.ds(gbase, BLKN)], sqv)
    lanes = lax.iota(jnp.int32, LANES)

    def row_body(li, cursor):
        xi = _splat(xv, li)
        yi = _splat(yv, li)
        zi = _splat(zv, li)
        sqi = _splat(sqv, li)
        gi = gbase + li
        gi_hi = jnp.broadcast_to(gi * 8192, (LANES,))

        def t_body(t, cur):
            jv = lanes + t * LANES
            xj = xv[pl.ds(t * LANES, LANES)]
            yj = yv[pl.ds(t * LANES, LANES)]
            zj = zv[pl.ds(t * LANES, LANES)]
            sqj = sqv[pl.ds(t * LANES, LANES)]
            xx = xi * xj
            yy = yi * yj
            zz = zi * zj
            s1, e1 = _two_sum(xx, yy)
            s2, e2 = _two_sum(s1, zz)
            mm = s2 + (e1 + e2)
            d = (sqi + sqj) - 2.0 * mm
            m = (d < C2) & (jv > li)
            off = jnp.minimum(cur, CAP - LANES)
            plsc.store_compressed(bpd.at[pl.ds(off, LANES)],
                                  gi_hi + (jv + gbase), mask=m)
            plsc.store_compressed(bd12.at[pl.ds(off, LANES)],
                                  jnp.maximum(d, 0.0), mask=m)
            cnt = plsc.all_reduce_population_count(m)[0]
            return jnp.minimum(cur + cnt, CAP - LANES)

        return lax.fori_loop(li // LANES, BLKN // LANES, t_body, cursor)

    cursor = lax.fori_loop(base_li, end_li, row_body, jnp.int32(0))
    cnt_v[...] = jnp.broadcast_to(cursor, (LANES,))
    pltpu.sync_copy(cnt_v, counts.at[w])
    pltpu.sync_copy(bpd, stage_pd.at[w])
    pltpu.sync_copy(bd12, stage_d12.at[w])


CH = 2048


def _phase2_body(spd, sd12, counts, esrc, edst, ed12,
                 cnt_vm, tpd, td12, gpd, gd12, osrc, odst, od12):
    cid = lax.axis_index("c")
    sid = lax.axis_index("s")
    w = sid * NC + cid
    pbase = w * OUT_PER_W
    pltpu.sync_copy(counts, cnt_vm)
    lanes = lax.iota(jnp.int32, LANES)
    zeros = jnp.zeros((LANES,), jnp.int32)

    total_v = zeros
    offs = []
    for s in range(W):
        cs = plsc.load_gather(cnt_vm, [jnp.full((LANES,), s, jnp.int32), zeros])
        offs.append(total_v[0])
        total_v = total_v + cs
    total = total_v[0]
    pe = jnp.minimum(total, pbase + OUT_PER_W)

    for s in range(W):
        a = jnp.maximum(offs[s], pbase)
        b = jnp.minimum(offs[s + 1] if s + 1 < W else total, pe)

        @pl.when(b > a)
        def _copy(s=s, a=a, b=b):
            srcoff = a - offs[s]
            sal = srcoff & ~7
            shift = srcoff - sal
            ln = (b - a) + shift
            base_flat = s * CAP + sal

            def chunk(cc, _):
                boff = pl.multiple_of(base_flat + cc * CH, 8)
                pltpu.sync_copy(spd.at[pl.ds(boff, CH)],
                                tpd.at[pl.ds(cc * CH, CH)])
                pltpu.sync_copy(sd12.at[pl.ds(boff, CH)],
                                td12.at[pl.ds(cc * CH, CH)])
                return 0

            lax.fori_loop(0, (ln + CH - 1) // CH, chunk, 0)
            dbase = a - pbase

            def shiftcp(v, _):
                gpd[pl.ds(dbase + v * LANES, LANES)] = (
                    tpd[pl.ds(shift + v * LANES, LANES)])
                gd12[pl.ds(dbase + v * LANES, LANES)] = (
                    td12[pl.ds(shift + v * LANES, LANES)])
                return 0

            lax.fori_loop(0, (b - a + LANES - 1) // LANES, shiftcp, 0)

    def post(t, _):
        sl = pl.ds(t * LANES, LANES)
        p = lanes + (pbase + t * LANES)
        v = p < total
        pd = gpd[sl]
        osrc[sl] = jnp.where(v, pd >> 13, N)
        odst[sl] = jnp.where(v, pd & 8191, N)
        od12[sl] = jnp.where(v, gd12[sl], C2)
        return 0

    lax.fori_loop(0, OUT_PER_W // LANES, post, 0)
    pltpu.sync_copy(osrc, esrc.at[pl.ds(pbase, OUT_PER_W)])
    pltpu.sync_copy(odst, esrc.at[pl.ds(M + pbase, OUT_PER_W)])
    pltpu.sync_copy(odst, edst.at[pl.ds(pbase, OUT_PER_W)])
    pltpu.sync_copy(osrc, edst.at[pl.ds(M + pbase, OUT_PER_W)])
    pltpu.sync_copy(od12, ed12.at[pl.ds(pbase, OUT_PER_W)])
    pltpu.sync_copy(od12, ed12.at[pl.ds(M + pbase, OUT_PER_W)])


def _bf16_round(x):
    y = lax.bitcast_convert_type(x, jnp.int32)
    r = (y + 0x7FFF + ((y >> 16) & 1)) & ~0xFFFF
    return lax.bitcast_convert_type(r, jnp.float32)


@functools.partial(jax.jit, static_argnames=())
def kernel(coordinates, batch_index, natoms):
    del batch_index, natoms
    c = coordinates
    sq = jnp.sum(c * c, axis=-1)
    ch = _bf16_round(c)
    xs = ch[:, 0]
    ys = ch[:, 1]
    zs = ch[:, 2]

    mesh = plsc.VectorSubcoreMesh(core_axis_name="c", subcore_axis_name="s")
    spd, sd12, counts = pl.kernel(
        _phase1_body,
        out_type=[
            jax.ShapeDtypeStruct((W + 1, CAP), jnp.int32),
            jax.ShapeDtypeStruct((W + 1, CAP), jnp.float32),
            jax.ShapeDtypeStruct((W, LANES), jnp.int32),
        ],
        mesh=mesh,
        compiler_params=pltpu.CompilerParams(needs_layout_passes=False),
        scratch_types=[
            pltpu.VMEM((BLKN,), jnp.float32),
            pltpu.VMEM((BLKN,), jnp.float32),
            pltpu.VMEM((BLKN,), jnp.float32),
            pltpu.VMEM((BLKN,), jnp.float32),
            pltpu.VMEM((CAP,), jnp.int32),
            pltpu.VMEM((CAP,), jnp.float32),
            pltpu.VMEM((LANES,), jnp.int32),
        ],
    )(xs, ys, zs, sq)

    esrc, edst, ed12 = pl.kernel(
        _phase2_body,
        out_type=[
            jax.ShapeDtypeStruct((2 * M,), jnp.int32),
            jax.ShapeDtypeStruct((2 * M,), jnp.int32),
            jax.ShapeDtypeStruct((2 * M,), jnp.float32),
        ],
        mesh=mesh,
        compiler_params=pltpu.CompilerParams(needs_layout_passes=False),
        scratch_types=[
            pltpu.VMEM((W, LANES), jnp.int32),
            pltpu.VMEM((5 * CH,), jnp.int32),
            pltpu.VMEM((5 * CH,), jnp.float32),
            pltpu.VMEM((OUT_PER_W + LANES,), jnp.int32),
            pltpu.VMEM((OUT_PER_W + LANES,), jnp.float32),
            pltpu.VMEM((OUT_PER_W,), jnp.int32),
            pltpu.VMEM((OUT_PER_W,), jnp.int32),
            pltpu.VMEM((OUT_PER_W,), jnp.float32),
        ],
    )(spd.reshape(-1), sd12.reshape(-1), counts)

    ie = jnp.arange(M, dtype=jnp.int32)
    isym = jnp.concatenate((ie + M, ie))
    return esrc, edst, ed12, isym
